# baseline (speedup 1.0000x reference)
import functools

import jax
import jax.numpy as jnp
from jax import lax
from jax.experimental import pallas as pl
from jax.experimental.pallas import tpu as pltpu
from jax.experimental.pallas import tpu_sc as plsc

BS = 1024
TSTPS = 50
N_NODES = 26
OUT_DIM = 64
TYPE_DIM = 3
TPAD = 8
ROWS = BS * TSTPS
PAIRW = 2 * OUT_DIM

_CHUNK = 128


def _sc_node_gather(node_pairs, pidx):
    info = plsc.get_sparse_core_info()
    nw = info.num_cores * info.num_subcores
    r_per_w = ROWS // nw
    half = r_per_w // 2

    mesh = plsc.VectorSubcoreMesh(core_axis_name="c", subcore_axis_name="s")

    @functools.partial(
        pl.kernel,
        mesh=mesh,
        out_type=jax.ShapeDtypeStruct((nw, r_per_w, PAIRW), jnp.float32),
        scratch_types=[
            pltpu.VMEM((r_per_w,), jnp.int32),
            pltpu.VMEM((half, PAIRW), jnp.float32),
            pltpu.SemaphoreType.DMA,
            pltpu.SemaphoreType.DMA,
        ],
    )
    def k(node_hbm, pidx_hbm, ng_out, idx_v, rows_v, sem, wsem):
        wid = lax.axis_index("s") * info.num_cores + lax.axis_index("c")
        pltpu.sync_copy(pidx_hbm.at[wid], idx_v)
        for h in range(2):
            copies = []
            for off in range(0, half, _CHUNK):
                sz = min(_CHUNK, half - off)
                copies.append(pltpu.async_copy(
                    node_hbm.at[idx_v.at[pl.ds(h * half + off, sz)]],
                    rows_v.at[pl.ds(off, sz)], sem))
            for c in copies:
                c.wait()
            pltpu.async_copy(
                rows_v, ng_out.at[wid, pl.ds(h * half, half)], wsem).wait()

    return k(node_pairs, pidx)


def _sc_type_gather(table, gid):
    info = plsc.get_sparse_core_info()
    nw = info.num_cores * info.num_subcores
    g_per_w = BS // nw

    mesh = plsc.VectorSubcoreMesh(core_axis_name="c", subcore_axis_name="s")

    @functools.partial(
        pl.kernel,
        mesh=mesh,
        compiler_params=pltpu.CompilerParams(use_tc_tiling_on_sc=False),
        out_type=jax.ShapeDtypeStruct((nw, g_per_w, TPAD), jnp.float32),
        scratch_types=[
            pltpu.VMEM((g_per_w,), jnp.int32),
            pltpu.VMEM((g_per_w, TPAD), jnp.float32),
            pltpu.SemaphoreType.DMA,
        ],
    )
    def k(tbl_hbm, gid_hbm, gt_out, gidx_v, gt_v, sem):
        wid = lax.axis_index("s") * info.num_cores + lax.axis_index("c")
        pltpu.sync_copy(gid_hbm.at[wid], gidx_v)
        pltpu.async_copy(tbl_hbm.at[gidx_v], gt_v, sem).wait()
        pltpu.sync_copy(gt_v, gt_out.at[wid])

    return k(table, gid)


_BB = 16
_BLK = _BB * TSTPS


def _tc_body(ng_ref, par_ref, gt_ref, s_ref, w2a_ref, w2b_ref, w1t_ref, b_ref,
             out_ref):
    ng = ng_ref[...]
    par = par_ref[...]
    gt = gt_ref[...]
    sel = s_ref[...]
    lo = jnp.dot(ng, w2a_ref[...], preferred_element_type=jnp.float32)
    hi = jnp.dot(ng, w2b_ref[...], preferred_element_type=jnp.float32)
    acc = lo + par * (hi - lo)
    gtr = jnp.dot(sel, gt, preferred_element_type=jnp.float32)
    acc += jnp.dot(gtr, w1t_ref[...], preferred_element_type=jnp.float32)
    out_ref[...] = acc + b_ref[...]


def _tc_fuse(ng, par, gt, w2a, w2b, w1t, bvec):
    sel = (lax.broadcasted_iota(jnp.int32, (_BLK, _BB), 0) // TSTPS ==
           lax.broadcasted_iota(jnp.int32, (_BLK, _BB), 1)).astype(jnp.float32)
    grid = (ROWS // _BLK,)
    return pl.pallas_call(
        _tc_body,
        grid=grid,
        in_specs=[
            pl.BlockSpec((_BLK, PAIRW), lambda i: (i, 0)),
            pl.BlockSpec((_BLK, 1), lambda i: (i, 0)),
            pl.BlockSpec((_BB, TPAD), lambda i: (i, 0)),
            pl.BlockSpec((_BLK, _BB), lambda i: (0, 0)),
            pl.BlockSpec((PAIRW, OUT_DIM), lambda i: (0, 0)),
            pl.BlockSpec((PAIRW, OUT_DIM), lambda i: (0, 0)),
            pl.BlockSpec((TPAD, OUT_DIM), lambda i: (0, 0)),
            pl.BlockSpec((1, OUT_DIM), lambda i: (0, 0)),
        ],
        out_specs=pl.BlockSpec((_BLK, OUT_DIM), lambda i: (i, 0)),
        out_shape=jax.ShapeDtypeStruct((ROWS, OUT_DIM), jnp.float32),
    )(ng, par, gt, sel, w2a, w2b, w1t, bvec)


def kernel(goal_id, goal_classnode_id, goal_node_id, node_repr, goal_type_table, W, b):
    del goal_classnode_id
    info = plsc.get_sparse_core_info()
    nw = info.num_cores * info.num_subcores

    node_pairs = node_repr.reshape(BS * TSTPS * N_NODES // 2, PAIRW)
    gnid = goal_node_id.astype(jnp.int32)
    row_idx = (
        jnp.arange(BS, dtype=jnp.int32)[:, None] * (TSTPS * N_NODES)
        + jnp.arange(TSTPS, dtype=jnp.int32)[None, :] * N_NODES
        + gnid[:, None]
    )
    pidx = (row_idx >> 1).reshape(nw, ROWS // nw)
    parity = jnp.broadcast_to(
        (gnid & 1).astype(jnp.float32)[:, None], (BS, TSTPS)).reshape(ROWS, 1)

    ng = _sc_node_gather(node_pairs, pidx).reshape(ROWS, PAIRW)

    table_pad = jnp.pad(goal_type_table, ((0, 0), (0, TPAD - TYPE_DIM)))
    gt = _sc_type_gather(
        table_pad, goal_id.astype(jnp.int32).reshape(nw, BS // nw)
    ).reshape(BS, TPAD)

    w2t = W[:, TYPE_DIM:].T
    zeros = jnp.zeros_like(w2t)
    w2a = jnp.concatenate([w2t, zeros], axis=0)
    w2b = jnp.concatenate([zeros, w2t], axis=0)
    w1t = jnp.pad(W[:, :TYPE_DIM].T, ((0, TPAD - TYPE_DIM), (0, 0)))
    out = _tc_fuse(ng, parity, gt, w2a, w2b, w1t, b.reshape(1, OUT_DIM))
    return out.reshape(BS, TSTPS, OUT_DIM)
